# baseline (device time: 24426 ns/iter reference)
import jax
import jax.numpy as jnp
from jax import lax
from jax.experimental import pallas as pl
from jax.experimental.pallas import tpu as pltpu


def kernel(Q, K, V):
    b, sq, h, d = Q.shape
    scale = d ** -0.5
    rows, cols = b * sq, h * d
    f32 = jnp.float32
    bf16 = jnp.bfloat16

    def body(q_ref, k_ref, v_ref, out_ref, ck_ref, cv_ref,
             k_send, k_recv, v_send, v_recv):
        my_x = lax.axis_index("x")
        my_y = lax.axis_index("y")
        my_z = lax.axis_index("z")
        partner = (my_x, 1 - my_y, my_z)

        barrier_sem = pltpu.get_barrier_semaphore()
        pl.semaphore_signal(
            barrier_sem, inc=1,
            device_id=partner, device_id_type=pl.DeviceIdType.MESH,
        )
        pl.semaphore_wait(barrier_sem, 1)

        rk = []
        rv = []
        for bi in range(b):
            sl = pl.ds(bi * sq, sq)
            rk.append(pltpu.make_async_remote_copy(
                src_ref=k_ref.at[sl], dst_ref=ck_ref.at[sl],
                send_sem=k_send.at[bi], recv_sem=k_recv.at[bi],
                device_id=partner, device_id_type=pl.DeviceIdType.MESH,
            ))
            rv.append(pltpu.make_async_remote_copy(
                src_ref=v_ref.at[sl], dst_ref=cv_ref.at[sl],
                send_sem=v_send.at[bi], recv_sem=v_recv.at[bi],
                device_id=partner, device_id_type=pl.DeviceIdType.MESH,
            ))
        rk[0].start()
        rv[0].start()
        rk[1].start()
        rv[1].start()

        dn = (((1,), (1,)), ((), ()))

        m1s, l1s, o1s = [], [], []
        for bi in range(b):
            for hi in range(h):
                r, c = bi * sq, hi * d
                q = q_ref[r:r + sq, c:c + d]
                k1 = k_ref[r:r + sq, c:c + d]
                v1 = v_ref[r:r + sq, c:c + d]
                s1 = lax.dot_general(q, k1, dn, preferred_element_type=f32)
                m1 = jnp.max(s1, axis=1, keepdims=True)
                p1 = jnp.exp(s1 - m1)
                l1 = jnp.sum(p1, axis=1, keepdims=True)
                o1 = jnp.dot(p1.astype(bf16), v1, preferred_element_type=f32)
                m1s.append(m1)
                l1s.append(l1)
                o1s.append(o1)

        for bi in range(b):
            rk[bi].wait_recv()
            s2s = []
            for hi in range(h):
                r, c = bi * sq, hi * d
                q = q_ref[r:r + sq, c:c + d]
                k2 = ck_ref[r:r + sq, c:c + d]
                s2s.append(lax.dot_general(q, k2, dn, preferred_element_type=f32))
            rv[bi].wait_recv()
            for hi in range(h):
                r, c = bi * sq, hi * d
                v2 = cv_ref[r:r + sq, c:c + d]
                s2 = s2s[hi]
                m1 = m1s[bi * h + hi]
                l1 = l1s[bi * h + hi]
                o1 = o1s[bi * h + hi]
                m2 = jnp.max(s2, axis=1, keepdims=True)
                p2 = jnp.exp(s2 - m2)
                l2 = jnp.sum(p2, axis=1, keepdims=True)
                o2 = jnp.dot(p2.astype(bf16), v2, preferred_element_type=f32)
                m = jnp.maximum(m1, m2)
                e1 = jnp.exp(m1 - m)
                e2 = jnp.exp(m2 - m)
                l = l1 * e1 + l2 * e2
                o = (o1 * e1 + o2 * e2) / l
                out_ref[r:r + sq, c:c + d] = o

        for bi in range(b):
            rk[bi].wait_send()
            rv[bi].wait_send()

    out2d = pl.pallas_call(
        body,
        out_shape=jax.ShapeDtypeStruct((rows, cols), f32),
        in_specs=[
            pl.BlockSpec(memory_space=pltpu.VMEM),
            pl.BlockSpec(memory_space=pltpu.VMEM),
            pl.BlockSpec(memory_space=pltpu.VMEM),
        ],
        out_specs=pl.BlockSpec(memory_space=pltpu.VMEM),
        scratch_shapes=[
            pltpu.VMEM((rows, cols), bf16),
            pltpu.VMEM((rows, cols), bf16),
            pltpu.SemaphoreType.DMA((b,)),
            pltpu.SemaphoreType.DMA((b,)),
            pltpu.SemaphoreType.DMA((b,)),
            pltpu.SemaphoreType.DMA((b,)),
        ],
        compiler_params=pltpu.CompilerParams(collective_id=0),
    )(
        (Q.reshape(rows, cols) * scale).astype(bf16),
        K.reshape(rows, cols).astype(bf16),
        V.reshape(rows, cols).astype(bf16),
    )
    return out2d.reshape(b, sq, h, d)


# device time: 22119 ns/iter; 1.1043x vs baseline; 1.1043x over previous
import jax
import jax.numpy as jnp
from jax import lax
from jax.experimental import pallas as pl
from jax.experimental.pallas import tpu as pltpu


def kernel(Q, K, V):
    b, sq, h, d = Q.shape
    scale = d ** -0.5
    rows, cols = b * sq, h * d
    f32 = jnp.float32
    bf16 = jnp.bfloat16

    def body(q_ref, k_ref, v_ref, out_ref, ck_ref, cv_ref,
             k_send, k_recv, v_send, v_recv):
        my_x = lax.axis_index("x")
        my_y = lax.axis_index("y")
        my_z = lax.axis_index("z")
        partner = (my_x, 1 - my_y, my_z)

        barrier_sem = pltpu.get_barrier_semaphore()
        pl.semaphore_signal(
            barrier_sem, inc=1,
            device_id=partner, device_id_type=pl.DeviceIdType.MESH,
        )
        pl.semaphore_wait(barrier_sem, 1)

        rk = []
        rv = []
        for bi in range(b):
            sl = pl.ds(bi * sq, sq)
            rk.append(pltpu.make_async_remote_copy(
                src_ref=k_ref.at[sl], dst_ref=ck_ref.at[sl],
                send_sem=k_send.at[bi], recv_sem=k_recv.at[bi],
                device_id=partner, device_id_type=pl.DeviceIdType.MESH,
            ))
            rv.append(pltpu.make_async_remote_copy(
                src_ref=v_ref.at[sl], dst_ref=cv_ref.at[sl],
                send_sem=v_send.at[bi], recv_sem=v_recv.at[bi],
                device_id=partner, device_id_type=pl.DeviceIdType.MESH,
            ))
        rk[0].start()
        rv[0].start()
        rk[1].start()
        rv[1].start()

        for bi in range(b):
            rk[bi].wait_recv()
            rv[bi].wait_recv()
        out_ref[...] = (ck_ref[...].astype(f32) + cv_ref[...].astype(f32))
        for bi in range(b):
            rk[bi].wait_send()
            rv[bi].wait_send()

    out2d = pl.pallas_call(
        body,
        out_shape=jax.ShapeDtypeStruct((rows, cols), f32),
        in_specs=[
            pl.BlockSpec(memory_space=pltpu.VMEM),
            pl.BlockSpec(memory_space=pltpu.VMEM),
            pl.BlockSpec(memory_space=pltpu.VMEM),
        ],
        out_specs=pl.BlockSpec(memory_space=pltpu.VMEM),
        scratch_shapes=[
            pltpu.VMEM((rows, cols), bf16),
            pltpu.VMEM((rows, cols), bf16),
            pltpu.SemaphoreType.DMA((b,)),
            pltpu.SemaphoreType.DMA((b,)),
            pltpu.SemaphoreType.DMA((b,)),
            pltpu.SemaphoreType.DMA((b,)),
        ],
        compiler_params=pltpu.CompilerParams(collective_id=0),
    )(
        (Q.reshape(rows, cols) * scale).astype(bf16),
        K.reshape(rows, cols).astype(bf16),
        V.reshape(rows, cols).astype(bf16),
    )
    return out2d.reshape(b, sq, h, d)


# device time: 19350 ns/iter; 1.2623x vs baseline; 1.1431x over previous
import jax
import jax.numpy as jnp
from jax import lax
from jax.experimental import pallas as pl
from jax.experimental.pallas import tpu as pltpu


def kernel(Q, K, V):
    b, sq, h, d = Q.shape
    scale = d ** -0.5
    hg = h // 2
    f32 = jnp.float32
    bf16 = jnp.bfloat16

    def body(q_ref, k_ref, v_ref, out_ref,
             ck, cv,
             k_send, k_recv, v_send, v_recv):
        my_x = lax.axis_index("x")
        my_y = lax.axis_index("y")
        my_z = lax.axis_index("z")
        partner = (my_x, 1 - my_y, my_z)

        barrier_sem = pltpu.get_barrier_semaphore()
        pl.semaphore_signal(
            barrier_sem, inc=1,
            device_id=partner, device_id_type=pl.DeviceIdType.MESH,
        )

        def rdma(buf, cbuf, send, recv, bi, g):
            sl = pl.ds(g * hg, hg)
            return pltpu.make_async_remote_copy(
                src_ref=buf.at[bi, sl], dst_ref=cbuf.at[bi, sl],
                send_sem=send.at[bi, g], recv_sem=recv.at[bi, g],
                device_id=partner, device_id_type=pl.DeviceIdType.MESH,
            )

        rk = [[rdma(k_ref, ck, k_send, k_recv, bi, g) for g in range(2)]
              for bi in range(b)]
        rv = [[rdma(v_ref, cv, v_send, v_recv, bi, g) for g in range(2)]
              for bi in range(b)]

        pl.semaphore_wait(barrier_sem, 1)
        for bi in range(b):
            for g in range(2):
                rk[bi][g].start()
                rv[bi][g].start()

        dn_tn = (((0,), (0,)), ((), ()))
        dn_nn = (((1,), (0,)), ((), ()))

        m1s, l1s, o1s = {}, {}, {}
        for bi in range(b):
            for hi in range(h):
                qh = q_ref[bi, hi]
                s1 = lax.dot_general(k_ref[bi, hi], qh, dn_tn,
                                     preferred_element_type=f32)
                m1 = jnp.max(s1, axis=0, keepdims=True)
                p1 = jnp.exp(s1 - m1)
                l1 = jnp.sum(p1, axis=0, keepdims=True)
                o1 = lax.dot_general(v_ref[bi, hi], p1.astype(bf16), dn_nn,
                                     preferred_element_type=f32)
                m1s[bi, hi] = m1
                l1s[bi, hi] = l1
                o1s[bi, hi] = o1

        for bi in range(b):
            for g in range(2):
                heads = range(g * hg, (g + 1) * hg)
                rk[bi][g].wait_recv()
                s2s = {}
                for hi in heads:
                    s2s[hi] = lax.dot_general(ck[bi, hi], q_ref[bi, hi], dn_tn,
                                              preferred_element_type=f32)
                rv[bi][g].wait_recv()
                for hi in heads:
                    s2 = s2s[hi]
                    m1, l1, o1 = m1s[bi, hi], l1s[bi, hi], o1s[bi, hi]
                    m2 = jnp.max(s2, axis=0, keepdims=True)
                    p2 = jnp.exp(s2 - m2)
                    l2 = jnp.sum(p2, axis=0, keepdims=True)
                    o2 = lax.dot_general(cv[bi, hi], p2.astype(bf16), dn_nn,
                                         preferred_element_type=f32)
                    m = jnp.maximum(m1, m2)
                    e1 = jnp.exp(m1 - m)
                    e2 = jnp.exp(m2 - m)
                    l = l1 * e1 + l2 * e2
                    out_ref[bi, hi] = ((o1 * e1 + o2 * e2) / l).astype(bf16)

        for bi in range(b):
            for g in range(2):
                rk[bi][g].wait_send()
                rv[bi][g].wait_send()

    out_t = pl.pallas_call(
        body,
        out_shape=jax.ShapeDtypeStruct((b, h, d, sq), bf16),
        in_specs=[
            pl.BlockSpec(memory_space=pltpu.VMEM),
            pl.BlockSpec(memory_space=pltpu.VMEM),
            pl.BlockSpec(memory_space=pltpu.VMEM),
        ],
        out_specs=pl.BlockSpec(memory_space=pltpu.VMEM),
        scratch_shapes=[
            pltpu.VMEM((b, h, d, sq), bf16),
            pltpu.VMEM((b, h, d, sq), bf16),
            pltpu.SemaphoreType.DMA((b, 2)),
            pltpu.SemaphoreType.DMA((b, 2)),
            pltpu.SemaphoreType.DMA((b, 2)),
            pltpu.SemaphoreType.DMA((b, 2)),
        ],
        compiler_params=pltpu.CompilerParams(collective_id=0),
    )(
        (jnp.transpose(Q, (0, 2, 3, 1)) * scale).astype(bf16),
        jnp.transpose(K, (0, 2, 3, 1)).astype(bf16),
        jnp.transpose(V, (0, 2, 3, 1)).astype(bf16),
    )
    return jnp.transpose(out_t, (0, 3, 1, 2)).astype(f32)
